# Initial kernel scaffold; baseline (speedup 1.0000x reference)
import jax
import jax.numpy as jnp
from jax.experimental import pallas as pl


def kernel(x, tables):
    raise NotImplementedError("write your pallas kernel here")


# speedup vs baseline: 4.7255x; 4.7255x over previous
import functools
import math

import jax
import jax.numpy as jnp
from jax import lax
from jax.experimental import pallas as pl
from jax.experimental.pallas import tpu as pltpu
from jax.experimental.pallas import tpu_sc as plsc

NUM_TABLES = 26
VOCAB = 100000
EMB_DIM = 32
OFFSET = 13
B, S = 4096, 50
N = B * S
SCALE = math.sqrt(EMB_DIM)

NC, NS = 2, 16
NW = NC * NS
P = 64
ROWS = NUM_TABLES * P
IDX_ROWS = ROWS // 128
POS_PER_W = N // NW
CHUNKS = POS_PER_W // P
D_OUT = OFFSET + NUM_TABLES * EMB_DIM
CONT_CHUNK = P * OFFSET
VEC_PER_POS = NUM_TABLES * EMB_DIM // 16


def _build_sc_kernel():
    mesh = plsc.VectorSubcoreMesh(core_axis_name="c", subcore_axis_name="s")

    @functools.partial(
        pl.kernel,
        mesh=mesh,
        out_type=jax.ShapeDtypeStruct((N, D_OUT), jnp.float32),
        compiler_params=pltpu.CompilerParams(use_tc_tiling_on_sc=False),
        scratch_types=[
            pltpu.VMEM((ROWS,), jnp.int32),
            pltpu.VMEM((ROWS, EMB_DIM), jnp.float32),
            pltpu.VMEM((P, D_OUT), jnp.float32),
            pltpu.VMEM((CONT_CHUNK + 16,), jnp.float32),
            pltpu.SemaphoreType.DMA,
        ],
    )
    def k(tables_hbm, idx_hbm, cont_hbm, out_hbm, idx_v, stage, out_v, cont_v, sem):
        wid = lax.axis_index("s") * NC + lax.axis_index("c")
        lane = lax.iota(jnp.int32, 16)
        is_cont = lane < OFFSET

        def chunk_body(c, carry):
            gpos = wid * POS_PER_W + c * P
            pltpu.sync_copy(idx_hbm.at[pl.ds(gpos * NUM_TABLES, ROWS)], idx_v)
            pltpu.sync_copy(
                cont_hbm.at[pl.ds(gpos * OFFSET, CONT_CHUNK)],
                cont_v.at[pl.ds(0, CONT_CHUNK)],
            )
            descs = [
                pltpu.async_copy(
                    tables_hbm.at[idx_v.at[pl.ds(r * 128, 128)]],
                    stage.at[pl.ds(r * 128, 128)],
                    sem,
                )
                for r in range(IDX_ROWS)
            ]
            for d in descs:
                d.wait()

            def pos_body(p, pcarry):
                for j in range(VEC_PER_POS):
                    v = stage[p * NUM_TABLES + j // 2, pl.ds((j % 2) * 16, 16)]
                    out_v[p, pl.ds(OFFSET + 16 * j, 16)] = v * SCALE
                cvec = cont_v[pl.ds(p * OFFSET, 16)]
                head = out_v[p, pl.ds(0, 16)]
                out_v[p, pl.ds(0, 16)] = jnp.where(is_cont, cvec, head)
                return pcarry

            lax.fori_loop(0, P, pos_body, 0)
            pltpu.sync_copy(out_v, out_hbm.at[pl.ds(gpos, P)])
            return carry

        lax.fori_loop(0, CHUNKS, chunk_body, 0)

    return k


def kernel(x, tables):
    x2 = x.reshape(N, OFFSET + NUM_TABLES)
    cont = x2[:, :OFFSET].reshape(-1)
    offs = jnp.arange(NUM_TABLES, dtype=jnp.int32) * VOCAB
    idx = (x2[:, OFFSET:].astype(jnp.int32) + offs[None, :]).reshape(N * NUM_TABLES)
    tflat = tables.reshape(NUM_TABLES * VOCAB, EMB_DIM)
    out = _build_sc_kernel()(tflat, idx, cont)
    return out.reshape(B, S, D_OUT)


# speedup vs baseline: 4.9269x; 1.0426x over previous
import functools
import math

import jax
import jax.numpy as jnp
from jax import lax
from jax.experimental import pallas as pl
from jax.experimental.pallas import tpu as pltpu
from jax.experimental.pallas import tpu_sc as plsc

NUM_TABLES = 26
VOCAB = 100000
EMB_DIM = 32
OFFSET = 13
B, S = 4096, 50
SCALE = math.sqrt(EMB_DIM)

NC, NS = 2, 16
NW = NC * NS
ROWS = NUM_TABLES * S
IDX_PAD = 1304
CONT_PAD = 656
B_PER_W = B // NW
D_OUT = OFFSET + NUM_TABLES * EMB_DIM
VEC_PER_POS = NUM_TABLES * EMB_DIM // 16
GATHER_STEP = 104


def _build_sc_kernel():
    mesh = plsc.VectorSubcoreMesh(core_axis_name="c", subcore_axis_name="s")

    @functools.partial(
        pl.kernel,
        mesh=mesh,
        out_type=jax.ShapeDtypeStruct((B, S, D_OUT), jnp.float32),
        compiler_params=pltpu.CompilerParams(use_tc_tiling_on_sc=False),
        scratch_types=[
            pltpu.VMEM((IDX_PAD,), jnp.int32),
            pltpu.VMEM((IDX_PAD, EMB_DIM), jnp.float32),
            pltpu.VMEM((S, D_OUT), jnp.float32),
            pltpu.VMEM((CONT_PAD + 16,), jnp.float32),
            pltpu.SemaphoreType.DMA,
        ],
    )
    def k(tables_hbm, idx_hbm, cont_hbm, out_hbm, idx_v, stage, out_v, cont_v, sem):
        wid = lax.axis_index("s") * NC + lax.axis_index("c")
        lane = lax.iota(jnp.int32, 16)
        is_cont = lane < OFFSET

        def row_body(i, carry):
            b = wid * B_PER_W + i
            pltpu.sync_copy(idx_hbm.at[pl.ds(b * IDX_PAD, IDX_PAD)], idx_v)
            pltpu.sync_copy(
                cont_hbm.at[pl.ds(b * CONT_PAD, CONT_PAD)],
                cont_v.at[pl.ds(0, CONT_PAD)],
            )
            descs = []
            off = 0
            while off < ROWS:
                step = min(GATHER_STEP, ROWS - off)
                descs.append(
                    pltpu.async_copy(
                        tables_hbm.at[idx_v.at[pl.ds(off, step)]],
                        stage.at[pl.ds(off, step)],
                        sem,
                    )
                )
                off += step
            for d in descs:
                d.wait()

            def pos_body(p, pcarry):
                for j in range(VEC_PER_POS):
                    v = stage[p * NUM_TABLES + j // 2, pl.ds((j % 2) * 16, 16)]
                    out_v[p, pl.ds(OFFSET + 16 * j, 16)] = v * SCALE
                cvec = cont_v[pl.ds(p * OFFSET, 16)]
                head = out_v[p, pl.ds(0, 16)]
                out_v[p, pl.ds(0, 16)] = jnp.where(is_cont, cvec, head)
                return pcarry

            lax.fori_loop(0, S, pos_body, 0)
            pltpu.sync_copy(out_v, out_hbm.at[b])
            return carry

        lax.fori_loop(0, B_PER_W, row_body, 0)

    return k


def kernel(x, tables):
    cont = x[:, :, :OFFSET].reshape(B, S * OFFSET)
    cont = jnp.pad(cont, ((0, 0), (0, CONT_PAD - S * OFFSET))).reshape(-1)
    offs = jnp.arange(NUM_TABLES, dtype=jnp.int32) * VOCAB
    idx = (x[:, :, OFFSET:].astype(jnp.int32) + offs).reshape(B, ROWS)
    idx = jnp.pad(idx, ((0, 0), (0, IDX_PAD - ROWS))).reshape(-1)
    tflat = tables.reshape(NUM_TABLES * VOCAB, EMB_DIM)
    return _build_sc_kernel()(tflat, idx, cont)


# speedup vs baseline: 5.0886x; 1.0328x over previous
import functools
import math

import jax
import jax.numpy as jnp
from jax import lax
from jax.experimental import pallas as pl
from jax.experimental.pallas import tpu as pltpu
from jax.experimental.pallas import tpu_sc as plsc

NUM_TABLES = 26
VOCAB = 100000
EMB_DIM = 32
OFFSET = 13
B, S = 4096, 50
SCALE = math.sqrt(EMB_DIM)

NC, NS = 2, 16
NW = NC * NS
ROWS = NUM_TABLES * S
S_PAD = 56
IDX_PAD = NUM_TABLES * S_PAD
B_PER_W = B // NW
D_EMB = NUM_TABLES * EMB_DIM
VEC_PER_POS = D_EMB // 16


def _build_sc_kernel():
    mesh = plsc.VectorSubcoreMesh(core_axis_name="c", subcore_axis_name="s")

    @functools.partial(
        pl.kernel,
        mesh=mesh,
        out_type=jax.ShapeDtypeStruct((B * S * D_EMB,), jnp.float32),
        compiler_params=pltpu.CompilerParams(use_tc_tiling_on_sc=False),
        scratch_types=[
            pltpu.VMEM((IDX_PAD,), jnp.int32),
            pltpu.VMEM((ROWS, EMB_DIM), jnp.float32),
            pltpu.VMEM((S * D_EMB,), jnp.float32),
            pltpu.SemaphoreType.DMA,
        ],
    )
    def k(tables_hbm, idx_hbm, out_hbm, idx_v, stage, out_v, sem):
        wid = lax.axis_index("s") * NC + lax.axis_index("c")

        def row_body(i, carry):
            b = wid * B_PER_W + i
            pltpu.sync_copy(idx_hbm.at[pl.ds(b * IDX_PAD, IDX_PAD)], idx_v)
            descs = [
                pltpu.async_copy(
                    tables_hbm.at[t].at[idx_v.at[pl.ds(t * S_PAD, S)]],
                    stage.at[pl.ds(t * S, S)],
                    sem,
                )
                for t in range(NUM_TABLES)
            ]
            for d in descs:
                d.wait()

            def pos_body(p, pcarry):
                for j in range(VEC_PER_POS):
                    v = stage[(j // 2) * S + p, pl.ds((j % 2) * 16, 16)]
                    out_v[pl.ds(p * D_EMB + 16 * j, 16)] = v * SCALE
                return pcarry

            lax.fori_loop(0, S, pos_body, 0)
            pltpu.sync_copy(
                out_v, out_hbm.at[pl.ds(b * S * D_EMB, S * D_EMB)]
            )
            return carry

        lax.fori_loop(0, B_PER_W, row_body, 0)

    return k


def kernel(x, tables):
    cont = x[:, :, :OFFSET]
    idx = x[:, :, OFFSET:].astype(jnp.int32)
    idx = jnp.swapaxes(idx, 1, 2)
    idx = jnp.pad(idx, ((0, 0), (0, 0), (0, S_PAD - S))).reshape(-1)
    emb = _build_sc_kernel()(tables, idx)
    emb = emb.reshape(B, S, D_EMB)
    return jnp.concatenate([cont, emb], axis=-1)
